# Initial kernel scaffold; baseline (speedup 1.0000x reference)
import jax
import jax.numpy as jnp
from jax.experimental import pallas as pl


def kernel(user_emb, item_emb, group_emb, u_rows, u_cols, u_vals, i_rows, i_cols, i_vals, f_rows, f_cols, f_vals, layers):
    raise NotImplementedError("write your pallas kernel here")


# speedup vs baseline: 1.5240x; 1.5240x over previous
import functools

import jax
import jax.numpy as jnp
from jax import lax
from jax.experimental import pallas as pl
from jax.experimental.pallas import tpu as pltpu
from jax.experimental.pallas import tpu_sc as plsc

_U, _I, _G, _D = 100000, 50000, 20000, 64
_N = _U + _I
_EU, _EI = 320000, 640000
_EF = _EU + _EI

_NC, _NS = 2, 16
_NW = _NC * _NS
_SL = 128
_KS = 8
_SB = _KS * _SL
_FCHUNK = 12512
_NP = 12 * _FCHUNK
_GCHUNK = _G // _NC
_TRASH = 1024


def _pad_up(e):
    m = _NS * _SB
    return ((e + m - 1) // m) * m


def _scale_rows(rows_v, val_v, j):
    def body(q, carry):
        vv = val_v[j, pl.ds(q * 16, 16)]
        base = q * 16
        for e in range(16):
            v = vv[e]
            for k in range(_D // 16):
                sl = pl.ds(k * 16, 16)
                rows_v[j, base + e, sl] = rows_v[j, base + e, sl] * v
        return carry
    lax.fori_loop(0, _SL // 16, body, 0)


def _make_spmm(e_pad, chunk, cpc, out_rows):
    per_tile = e_pad // _NS
    nsb = per_tile // _SB
    rpt = per_tile // _SL
    wz = ((chunk // _NS) + 7) // 8 * 8
    wz_last = chunk - 15 * wz

    mesh = plsc.VectorSubcoreMesh(core_axis_name="c", subcore_axis_name="s")

    @functools.partial(
        pl.kernel,
        out_type=jax.ShapeDtypeStruct((out_rows, _D), jnp.float32),
        mesh=mesh,
        compiler_params=pltpu.CompilerParams(use_tc_tiling_on_sc=False),
        scratch_types=[
            pltpu.VMEM((_KS, _SL), jnp.int32),
            pltpu.VMEM((_KS, _SL), jnp.int32),
            pltpu.VMEM((_KS, _SL), jnp.float32),
            pltpu.VMEM((_KS, _SL, _D), jnp.float32),
            pltpu.VMEM_SHARED((chunk + _TRASH, _D), jnp.float32),
            pltpu.SemaphoreType.DMA,
        ],
    )
    def spmm(table, cols2, segs2, vals2, zeros, out, idx_v, seg_v, val_v,
             rows_v, acc, sem):
        c = lax.axis_index("c")
        s = lax.axis_index("s")
        row_base = s * rpt
        iota = lax.broadcasted_iota(jnp.int32, (16,), 0)

        def one_pass(p, carry):
            base = (c * cpc + p) * chunk

            @pl.when(s < 15)
            def _():
                pltpu.sync_copy(zeros.at[pl.ds(s * wz, wz)],
                                acc.at[pl.ds(s * wz, wz)])

            @pl.when(s == 15)
            def _():
                pltpu.sync_copy(zeros.at[pl.ds(15 * wz, wz_last)],
                                acc.at[pl.ds(15 * wz, wz_last)])

            plsc.subcore_barrier()

            def sb_body(b, carry2):
                r0 = row_base + b * _KS
                pltpu.sync_copy(cols2.at[pl.ds(r0, _KS)], idx_v)
                pltpu.sync_copy(segs2.at[pl.ds(r0, _KS)], seg_v)
                pltpu.sync_copy(vals2.at[pl.ds(r0, _KS)], val_v)
                cps = [pltpu.async_copy(table.at[idx_v.at[j]], rows_v.at[j],
                                        sem) for j in range(_KS)]
                for cp in cps:
                    cp.wait()
                for j in range(_KS):
                    _scale_rows(rows_v, val_v, j)
                    for q in range(_SL // 16):
                        sl = pl.ds(q * 16, 16)
                        loc = seg_v[j, sl] - base
                        pos = j * _SL + q * 16
                        tr = chunk + ((iota + (pos + s * 64)) & (_TRASH - 1))
                        oob = (loc < 0) | (loc >= chunk)
                        seg_v[j, sl] = jnp.where(oob, tr, loc)
                for j in range(_KS):
                    pltpu.sync_copy(rows_v.at[j], acc.at[seg_v.at[j]],
                                    add=True)
                return carry2

            lax.fori_loop(0, nsb, sb_body, 0)
            plsc.subcore_barrier()

            @pl.when(s < 15)
            def _():
                pltpu.sync_copy(acc.at[pl.ds(s * wz, wz)],
                                out.at[pl.ds(base + s * wz, wz)])

            @pl.when(s == 15)
            def _():
                pltpu.sync_copy(acc.at[pl.ds(15 * wz, wz_last)],
                                out.at[pl.ds(base + 15 * wz, wz_last)])

            plsc.subcore_barrier()
            return carry

        lax.fori_loop(0, cpc, one_pass, 0)

    return spmm


def _dense_layer(um, im, init_g, wq1, bq1, wq2, wu, bu, wi, bi):
    bg = 2000
    f32 = jnp.float32

    def body(um_ref, im_ref, g_ref, wq1_ref, bq1_ref, wq2_ref, wu_ref,
             bu_ref, wi_ref, bi_ref, msg_ref):
        u = um_ref[...]
        i = im_ref[...]
        w1 = wq1_ref[...]
        b1 = bq1_ref[...]
        w2 = wq2_ref[...]
        tu = jnp.tanh(jnp.dot(u, w1, preferred_element_type=f32) + b1)
        ti = jnp.tanh(jnp.dot(i, w1, preferred_element_type=f32) + b1)
        au = jnp.sum(tu * w2, axis=1, keepdims=True)
        ai = jnp.sum(ti * w2, axis=1, keepdims=True)
        w0 = jax.nn.sigmoid(au - ai)
        common = w0 * u + (1.0 - w0) * i
        g = g_ref[...]
        mu = (jnp.dot(u - common, wu_ref[0:_D, :], preferred_element_type=f32)
              + jnp.dot(g, wu_ref[_D:2 * _D, :], preferred_element_type=f32)
              + bu_ref[...])
        mi = (jnp.dot(i - common, wi_ref[0:_D, :], preferred_element_type=f32)
              + jnp.dot(g, wi_ref[_D:2 * _D, :], preferred_element_type=f32)
              + bi_ref[...])
        msg_ref[...] = mu + mi + common

    def wspec(shape):
        return pl.BlockSpec(shape, lambda i: tuple(0 for _ in shape))

    gspec = pl.BlockSpec((bg, _D), lambda i: (i, 0))
    return pl.pallas_call(
        body,
        grid=(_G // bg,),
        in_specs=[
            gspec, gspec, gspec,
            wspec((_D, _D)), wspec((1, _D)), wspec((1, _D)),
            wspec((2 * _D, _D)), wspec((1, _D)),
            wspec((2 * _D, _D)), wspec((1, _D)),
        ],
        out_specs=gspec,
        out_shape=jax.ShapeDtypeStruct((_G, _D), jnp.float32),
    )(um, im, init_g, wq1, bq1, wq2, wu, bu, wi, bi)


def _add4(rows, bs):
    def body(a_ref, b_ref, c_ref, d_ref, o_ref):
        o_ref[...] = a_ref[...] + b_ref[...] + c_ref[...] + d_ref[...]

    spec = pl.BlockSpec((bs, _D), lambda i: (i, 0))
    return pl.pallas_call(
        body,
        grid=(rows // bs,),
        in_specs=[spec, spec, spec, spec],
        out_specs=spec,
        out_shape=jax.ShapeDtypeStruct((rows, _D), jnp.float32),
    )


def _prep_edges(segs, cols, vals, e_pad, col_mod, col_off):
    e = segs.shape[0]
    pad = e_pad - e
    pidx = jnp.arange(pad, dtype=jnp.int32)
    segs_p = jnp.concatenate([segs, pidx % jnp.int32(16384)])
    cols_p = jnp.concatenate(
        [cols + jnp.int32(col_off),
         (pidx % jnp.int32(col_mod)) + jnp.int32(col_off)])
    vals_p = jnp.concatenate([vals, jnp.zeros((pad,), jnp.float32)])
    r = e_pad // _SL
    return (cols_p.reshape(r, _SL), segs_p.reshape(r, _SL),
            vals_p.reshape(r, _SL))


def kernel(user_emb, item_emb, group_emb, u_rows, u_cols, u_vals,
           i_rows, i_cols, i_vals, f_rows, f_cols, f_vals, layers):
    init_ui = jnp.concatenate(
        [user_emb, item_emb,
         jnp.zeros((_NP - _N, _D), jnp.float32)], axis=0)
    zeros = jnp.zeros((_FCHUNK, _D), jnp.float32)

    eu_p, ei_p, ef_p = _pad_up(_EU), _pad_up(_EI), _pad_up(_EF)
    u_c2, u_s2, u_v2 = _prep_edges(u_rows, u_cols, u_vals, eu_p, 65536, 0)
    i_c2, i_s2, i_v2 = _prep_edges(i_rows, i_cols, i_vals, ei_p, 32768, _U)
    f_c2, f_s2, f_v2 = _prep_edges(f_rows, f_cols, f_vals, ef_p, 16384, 0)

    spmm_u = _make_spmm(eu_p, _GCHUNK, 1, _G)
    spmm_i = _make_spmm(ei_p, _GCHUNK, 1, _G)
    spmm_f = _make_spmm(ef_p, _FCHUNK, 6, _NP)

    table = init_ui
    nodes, msgs = [], []
    for p in layers:
        um = spmm_u(table, u_c2, u_s2, u_v2, zeros)
        im = spmm_i(table, i_c2, i_s2, i_v2, zeros)
        msg = _dense_layer(um, im, group_emb, p["Wq1"],
                           p["bq1"].reshape(1, _D), p["Wq2"].reshape(1, _D),
                           p["Wu"], p["bu"].reshape(1, _D),
                           p["Wi"], p["bi"].reshape(1, _D))
        node = spmm_f(msg, f_c2, f_s2, f_v2, zeros)
        nodes.append(node)
        msgs.append(msg)
        table = node

    final_emb = _add4(_NP, 3128)(init_ui, *nodes)
    he_emb = _add4(_G, 2000)(group_emb, *msgs)
    return jnp.concatenate([final_emb[:_N], he_emb], axis=0)


# speedup vs baseline: 4.7436x; 3.1126x over previous
import functools

import jax
import jax.numpy as jnp
from jax import lax
from jax.experimental import pallas as pl
from jax.experimental.pallas import tpu as pltpu
from jax.experimental.pallas import tpu_sc as plsc

_U, _I, _G, _D = 100000, 50000, 20000, 64
_N = _U + _I
_EU, _EI = 320000, 640000
_EF = _EU + _EI

_NC, _NS = 2, 16
_NW = _NC * _NS
_SL = 128
_KS = 8
_SB = _KS * _SL
_FCHUNK = 8192
_NB_F = 20
_NP = _NB_F * _FCHUNK
_GCHUNK = _G // _NC


def _pad_up(e):
    return ((e + _NW * _SB - 1) // (_NW * _SB)) * (_NW * _SB)


def _make_partition(e_pad, nb, bucket_shift):
    per_w = e_pad // _NW
    cap = per_w + _SB
    ngrp = per_w // 16
    st_len = _SB + 16

    mesh = plsc.VectorSubcoreMesh(core_axis_name="c", subcore_axis_name="s")

    @functools.partial(
        pl.kernel,
        out_type=[
            jax.ShapeDtypeStruct((_NW, nb, cap), jnp.int32),
            jax.ShapeDtypeStruct((_NW, nb, cap), jnp.int32),
            jax.ShapeDtypeStruct((_NW, nb, cap), jnp.float32),
            jax.ShapeDtypeStruct((_NW, 32), jnp.int32),
        ],
        mesh=mesh,
        compiler_params=pltpu.CompilerParams(
            use_tc_tiling_on_sc=False, needs_layout_passes=False),
        scratch_types=[
            pltpu.VMEM((_KS, _SL), jnp.int32),
            pltpu.VMEM((_KS, _SL), jnp.int32),
            pltpu.VMEM((_KS, _SL), jnp.float32),
            pltpu.VMEM((nb * st_len,), jnp.int32),
            pltpu.VMEM((nb * st_len,), jnp.int32),
            pltpu.VMEM((nb * st_len,), jnp.float32),
            pltpu.VMEM((32,), jnp.int32),
            pltpu.SMEM((2 * nb,), jnp.int32),
            pltpu.SemaphoreType.DMA,
        ],
    )
    def part(cols2, segs2, vals2, cols_o, segs_o, vals_o, counts_o,
             colv, segv, valv, st_c, st_s, st_v, cnt_v, offs, sem):
        c = lax.axis_index("c")
        s = lax.axis_index("s")
        w = s * _NC + c
        row_base = w * (per_w // _SL)
        iota = lax.broadcasted_iota(jnp.int32, (16,), 0)
        zf = jnp.zeros((16,), jnp.float32)

        def init_b(z, carry):
            for b in range(nb):
                st_v[pl.ds(b * st_len + z * 16, 16)] = zf
                st_c[pl.ds(b * st_len + z * 16, 16)] = (iota + z) & 8191
                st_s[pl.ds(b * st_len + z * 16, 16)] = (
                    jnp.zeros((16,), jnp.int32) + b * _GCHUNK
                    if bucket_shift == 0 else
                    jnp.zeros((16,), jnp.int32) + b * (1 << bucket_shift))
            return carry
        lax.fori_loop(0, st_len // 16, init_b, 0)
        for b in range(2 * nb):
            offs[b] = 0

        def flush_block(b, off):
            tot = pl.multiple_of(offs[nb + b], _SB)
            pltpu.sync_copy(st_c.at[pl.ds(b * st_len, _SB)],
                            cols_o.at[w, b, pl.ds(tot, _SB)])
            pltpu.sync_copy(st_s.at[pl.ds(b * st_len, _SB)],
                            segs_o.at[w, b, pl.ds(tot, _SB)])
            pltpu.sync_copy(st_v.at[pl.ds(b * st_len, _SB)],
                            vals_o.at[w, b, pl.ds(tot, _SB)])
            mc = st_c[pl.ds(b * st_len + _SB, 16)]
            ms = st_s[pl.ds(b * st_len + _SB, 16)]
            mv = st_v[pl.ds(b * st_len + _SB, 16)]
            rem = off - _SB
            mv = jnp.where(iota < rem, mv, 0.0)

            def zv(z, cz):
                st_v[pl.ds(b * st_len + z * 16, 16)] = zf
                return cz
            lax.fori_loop(0, st_len // 16, zv, 0)
            st_c[pl.ds(b * st_len, 16)] = mc
            st_s[pl.ds(b * st_len, 16)] = ms
            st_v[pl.ds(b * st_len, 16)] = mv
            offs[b] = rem
            offs[nb + b] = tot + _SB

        def sb_body(sb, carry):
            r0 = row_base + sb * _KS
            pltpu.sync_copy(cols2.at[pl.ds(r0, _KS)], colv)
            pltpu.sync_copy(segs2.at[pl.ds(r0, _KS)], segv)
            pltpu.sync_copy(vals2.at[pl.ds(r0, _KS)], valv)

            def grp(g, carry2):
                gj = g >> 3
                gq = (g & 7) * 16
                sg = segv[gj, pl.ds(gq, 16)]
                cg = colv[gj, pl.ds(gq, 16)]
                vg = valv[gj, pl.ds(gq, 16)]
                if bucket_shift == 0:
                    bv = (sg >= _GCHUNK).astype(jnp.int32)
                else:
                    bv = jnp.right_shift(sg, jnp.int32(bucket_shift))
                for b in range(nb):
                    off = offs[b]

                    @pl.when(off >= _SB)
                    def _(b=b, off=off):
                        flush_block(b, off)

                    off2 = offs[b]
                    m = bv == b
                    plsc.store_compressed(
                        st_c.at[pl.ds(b * st_len + off2, 16)], cg, mask=m)
                    plsc.store_compressed(
                        st_s.at[pl.ds(b * st_len + off2, 16)], sg, mask=m)
                    plsc.store_compressed(
                        st_v.at[pl.ds(b * st_len + off2, 16)], vg, mask=m)
                    pc = plsc.all_reduce_population_count(m)
                    offs[b] = off2 + pc[0]
                return carry2

            lax.fori_loop(0, _KS * _SL // 16, grp, 0)
            return carry

        lax.fori_loop(0, per_w // _SB, sb_body, 0)

        for b in range(nb):
            offp = offs[b]

            @pl.when(offp >= _SB)
            def _(b=b, offp=offp):
                flush_block(b, offp)

        cz16 = jnp.zeros((16,), jnp.int32)
        clo = cz16
        chi = cz16
        for b in range(nb):
            tot = pl.multiple_of(offs[nb + b], _SB)
            pltpu.sync_copy(st_c.at[pl.ds(b * st_len, _SB)],
                            cols_o.at[w, b, pl.ds(tot, _SB)])
            pltpu.sync_copy(st_s.at[pl.ds(b * st_len, _SB)],
                            segs_o.at[w, b, pl.ds(tot, _SB)])
            pltpu.sync_copy(st_v.at[pl.ds(b * st_len, _SB)],
                            vals_o.at[w, b, pl.ds(tot, _SB)])
            cnt = tot + offs[b]
            if b < 16:
                clo = jnp.where(iota == b, cnt, clo)
            else:
                chi = jnp.where(iota == (b - 16), cnt, chi)
        cnt_v[pl.ds(0, 16)] = clo
        cnt_v[pl.ds(16, 16)] = chi
        pltpu.sync_copy(cnt_v, counts_o.at[w])

    return part


def _scale_rows(rows_v, val_v, j):
    def body(q, carry):
        vv = val_v[j, pl.ds(q * 16, 16)]
        base = q * 16
        for e in range(16):
            v = vv[e]
            for k in range(_D // 16):
                sl = pl.ds(k * 16, 16)
                rows_v[j, base + e, sl] = rows_v[j, base + e, sl] * v
        return carry
    lax.fori_loop(0, _SL // 16, body, 0)


def _make_spmm(e_pad, chunk, cpc, nb, out_rows):
    cap = e_pad // _NW + _SB
    wz = ((chunk // _NS) + 7) // 8 * 8
    wz_last = chunk - 15 * wz

    mesh = plsc.VectorSubcoreMesh(core_axis_name="c", subcore_axis_name="s")

    @functools.partial(
        pl.kernel,
        out_type=jax.ShapeDtypeStruct((out_rows, _D), jnp.float32),
        mesh=mesh,
        compiler_params=pltpu.CompilerParams(
            use_tc_tiling_on_sc=False, needs_layout_passes=False),
        scratch_types=[
            pltpu.VMEM((_KS, _SL), jnp.int32),
            pltpu.VMEM((_KS, _SL), jnp.int32),
            pltpu.VMEM((_KS, _SL), jnp.float32),
            pltpu.VMEM((_KS, _SL, _D), jnp.float32),
            pltpu.VMEM((64,), jnp.int32),
            pltpu.VMEM_SHARED((chunk, _D), jnp.float32),
            pltpu.SemaphoreType.DMA,
        ],
    )
    def spmm(table, cols5, segs5, vals5, counts, zeros, out, idx_v, seg_v,
             val_v, rows_v, cv, acc, sem):
        c = lax.axis_index("c")
        s = lax.axis_index("s")
        iota = lax.broadcasted_iota(jnp.int32, (16,), 0)
        pltpu.sync_copy(counts.at[2 * s], cv.at[pl.ds(0, 32)])
        pltpu.sync_copy(counts.at[2 * s + 1], cv.at[pl.ds(32, 32)])

        def one_bucket(bl, carry):
            b = c * cpc + bl

            @pl.when(s < 15)
            def _():
                pltpu.sync_copy(zeros.at[pl.ds(s * wz, wz)],
                                acc.at[pl.ds(s * wz, wz)])

            @pl.when(s == 15)
            def _():
                pltpu.sync_copy(zeros.at[pl.ds(15 * wz, wz_last)],
                                acc.at[pl.ds(15 * wz, wz_last)])

            plsc.subcore_barrier()
            base = b * chunk

            for wi in range(2):
                w = 2 * s + wi
                lo = cv[pl.ds(wi * 32, 16)]
                if cpc == 1:
                    cnt = jnp.where(c == 0, lo[0], lo[1])
                else:
                    cnt = jnp.sum(jnp.where(iota == b, lo, 0))
                    if nb > 16:
                        hi = cv[pl.ds(wi * 32 + 16, 16)]
                        cnt = cnt + jnp.sum(
                            jnp.where(iota == (b - 16), hi, 0))
                nsb = (cnt + _SB - 1) >> 10

                def sb_body(sb, carry2):
                    @pl.when(sb < nsb)
                    def _():
                        r0 = ((w * nb + b) * (cap // _SL)) + sb * _KS
                        pltpu.sync_copy(cols5.at[pl.ds(r0, _KS)], idx_v)
                        pltpu.sync_copy(segs5.at[pl.ds(r0, _KS)], seg_v)
                        pltpu.sync_copy(vals5.at[pl.ds(r0, _KS)], val_v)
                        cps = [pltpu.async_copy(table.at[idx_v.at[j]],
                                                rows_v.at[j], sem)
                               for j in range(_KS)]
                        for cp in cps:
                            cp.wait()
                        for j in range(_KS):
                            _scale_rows(rows_v, val_v, j)
                            for q in range(_SL // 16):
                                sl = pl.ds(q * 16, 16)
                                seg_v[j, sl] = seg_v[j, sl] - base
                        for j in range(_KS):
                            pltpu.sync_copy(rows_v.at[j], acc.at[seg_v.at[j]],
                                            add=True)
                    return carry2

                lax.fori_loop(0, cap // _SB, sb_body, 0)

            plsc.subcore_barrier()

            @pl.when(s < 15)
            def _():
                pltpu.sync_copy(acc.at[pl.ds(s * wz, wz)],
                                out.at[pl.ds(base + s * wz, wz)])

            @pl.when(s == 15)
            def _():
                pltpu.sync_copy(acc.at[pl.ds(15 * wz, wz_last)],
                                out.at[pl.ds(base + 15 * wz, wz_last)])

            plsc.subcore_barrier()
            return carry

        lax.fori_loop(0, cpc, one_bucket, 0)

    return spmm


def _dense_layer(um, im, init_g, wq1, bq1, wq2, wu, bu, wi, bi):
    bg = 2000
    f32 = jnp.float32

    def body(um_ref, im_ref, g_ref, wq1_ref, bq1_ref, wq2_ref, wu_ref,
             bu_ref, wi_ref, bi_ref, msg_ref):
        u = um_ref[...]
        i = im_ref[...]
        w1 = wq1_ref[...]
        b1 = bq1_ref[...]
        w2 = wq2_ref[...]
        tu = jnp.tanh(jnp.dot(u, w1, preferred_element_type=f32) + b1)
        ti = jnp.tanh(jnp.dot(i, w1, preferred_element_type=f32) + b1)
        au = jnp.sum(tu * w2, axis=1, keepdims=True)
        ai = jnp.sum(ti * w2, axis=1, keepdims=True)
        w0 = jax.nn.sigmoid(au - ai)
        common = w0 * u + (1.0 - w0) * i
        g = g_ref[...]
        mu = (jnp.dot(u - common, wu_ref[0:_D, :], preferred_element_type=f32)
              + jnp.dot(g, wu_ref[_D:2 * _D, :], preferred_element_type=f32)
              + bu_ref[...])
        mi = (jnp.dot(i - common, wi_ref[0:_D, :], preferred_element_type=f32)
              + jnp.dot(g, wi_ref[_D:2 * _D, :], preferred_element_type=f32)
              + bi_ref[...])
        msg_ref[...] = mu + mi + common

    def wspec(shape):
        return pl.BlockSpec(shape, lambda i: tuple(0 for _ in shape))

    gspec = pl.BlockSpec((bg, _D), lambda i: (i, 0))
    return pl.pallas_call(
        body,
        grid=(_G // bg,),
        in_specs=[
            gspec, gspec, gspec,
            wspec((_D, _D)), wspec((1, _D)), wspec((1, _D)),
            wspec((2 * _D, _D)), wspec((1, _D)),
            wspec((2 * _D, _D)), wspec((1, _D)),
        ],
        out_specs=gspec,
        out_shape=jax.ShapeDtypeStruct((_G, _D), jnp.float32),
    )(um, im, init_g, wq1, bq1, wq2, wu, bu, wi, bi)


def _add4(rows, bs):
    def body(a_ref, b_ref, c_ref, d_ref, o_ref):
        o_ref[...] = a_ref[...] + b_ref[...] + c_ref[...] + d_ref[...]

    spec = pl.BlockSpec((bs, _D), lambda i: (i, 0))
    return pl.pallas_call(
        body,
        grid=(rows // bs,),
        in_specs=[spec, spec, spec, spec],
        out_specs=spec,
        out_shape=jax.ShapeDtypeStruct((rows, _D), jnp.float32),
    )


def _prep_edges(segs, cols, vals, e_pad, col_mod, col_off):
    e = segs.shape[0]
    pad = e_pad - e
    pidx = jnp.arange(pad, dtype=jnp.int32)
    segs_p = jnp.concatenate([segs, pidx % jnp.int32(8192)])
    cols_p = jnp.concatenate(
        [cols + jnp.int32(col_off),
         (pidx % jnp.int32(col_mod)) + jnp.int32(col_off)])
    vals_p = jnp.concatenate([vals, jnp.zeros((pad,), jnp.float32)])
    r = e_pad // _SL
    return (cols_p.reshape(r, _SL), segs_p.reshape(r, _SL),
            vals_p.reshape(r, _SL))


def _bucketed(part_fn, cols2, segs2, vals2, e_pad, nb):
    cb, sb, vb, cnts = part_fn(cols2, segs2, vals2)
    cap = e_pad // _NW + _SB
    shp = (_NW * nb * (cap // _SL), _SL)
    return cb.reshape(shp), sb.reshape(shp), vb.reshape(shp), cnts


def kernel(user_emb, item_emb, group_emb, u_rows, u_cols, u_vals,
           i_rows, i_cols, i_vals, f_rows, f_cols, f_vals, layers):
    init_ui = jnp.concatenate(
        [user_emb, item_emb,
         jnp.zeros((_NP - _N, _D), jnp.float32)], axis=0)
    zeros = jnp.zeros((_GCHUNK, _D), jnp.float32)

    eu_p, ei_p, ef_p = _pad_up(_EU), _pad_up(_EI), _pad_up(_EF)
    u_c2, u_s2, u_v2 = _prep_edges(u_rows, u_cols, u_vals, eu_p, 65536, 0)
    i_c2, i_s2, i_v2 = _prep_edges(i_rows, i_cols, i_vals, ei_p, 32768, _U)
    f_c2, f_s2, f_v2 = _prep_edges(f_rows, f_cols, f_vals, ef_p, 8192, 0)

    u_bkt = _bucketed(_make_partition(eu_p, 2, 0), u_c2, u_s2, u_v2, eu_p, 2)
    i_bkt = _bucketed(_make_partition(ei_p, 2, 0), i_c2, i_s2, i_v2, ei_p, 2)
    f_bkt = _bucketed(_make_partition(ef_p, _NB_F, 13), f_c2, f_s2, f_v2,
                      ef_p, _NB_F)

    spmm_u = _make_spmm(eu_p, _GCHUNK, 1, 2, _G)
    spmm_i = _make_spmm(ei_p, _GCHUNK, 1, 2, _G)
    spmm_f = _make_spmm(ef_p, _FCHUNK, _NB_F // _NC, _NB_F, _NP)

    table = init_ui
    nodes, msgs = [], []
    for p in layers:
        um = spmm_u(table, *u_bkt, zeros)
        im = spmm_i(table, *i_bkt, zeros)
        msg = _dense_layer(um, im, group_emb, p["Wq1"],
                           p["bq1"].reshape(1, _D), p["Wq2"].reshape(1, _D),
                           p["Wu"], p["bu"].reshape(1, _D),
                           p["Wi"], p["bi"].reshape(1, _D))
        node = spmm_f(msg, *f_bkt, zeros)
        nodes.append(node)
        msgs.append(msg)
        table = node

    final_emb = _add4(_NP, 2560)(init_ui, *nodes)
    he_emb = _add4(_G, 2000)(group_emb, *msgs)
    return jnp.concatenate([final_emb[:_N], he_emb], axis=0)


# speedup vs baseline: 6.5122x; 1.3728x over previous
import functools

import jax
import jax.numpy as jnp
from jax import lax
from jax.experimental import pallas as pl
from jax.experimental.pallas import tpu as pltpu
from jax.experimental.pallas import tpu_sc as plsc

_U, _I, _G, _D = 100000, 50000, 20000, 64
_N = _U + _I
_EU, _EI = 320000, 640000
_EF = _EU + _EI

_NC, _NS = 2, 16
_NW = _NC * _NS
_SL = 128
_KS = 8
_SB = _KS * _SL
_FCHUNK = 8192
_NB_F = 20
_NP = _NB_F * _FCHUNK
_GCHUNK = _G // _NC


def _pad_up(e):
    return ((e + _NW * _SB - 1) // (_NW * _SB)) * (_NW * _SB)


def _make_partition(e_pad, nb, bucket_shift):
    per_w = e_pad // _NW
    cap = per_w + _SB
    ngrp = per_w // 16
    st_len = _SB + 16

    mesh = plsc.VectorSubcoreMesh(core_axis_name="c", subcore_axis_name="s")

    @functools.partial(
        pl.kernel,
        out_type=[
            jax.ShapeDtypeStruct((_NW, nb, cap), jnp.int32),
            jax.ShapeDtypeStruct((_NW, nb, cap), jnp.int32),
            jax.ShapeDtypeStruct((_NW, nb, cap), jnp.float32),
            jax.ShapeDtypeStruct((_NW, 32), jnp.int32),
        ],
        mesh=mesh,
        compiler_params=pltpu.CompilerParams(
            use_tc_tiling_on_sc=False, needs_layout_passes=False),
        scratch_types=[
            pltpu.VMEM((_KS, _SL), jnp.int32),
            pltpu.VMEM((_KS, _SL), jnp.int32),
            pltpu.VMEM((_KS, _SL), jnp.float32),
            pltpu.VMEM((nb * st_len,), jnp.int32),
            pltpu.VMEM((nb * st_len,), jnp.int32),
            pltpu.VMEM((nb * st_len,), jnp.float32),
            pltpu.VMEM((32,), jnp.int32),
            pltpu.SMEM((2 * nb,), jnp.int32),
            pltpu.SemaphoreType.DMA,
        ],
    )
    def part(cols2, segs2, vals2, cols_o, segs_o, vals_o, counts_o,
             colv, segv, valv, st_c, st_s, st_v, cnt_v, offs, sem):
        c = lax.axis_index("c")
        s = lax.axis_index("s")
        w = s * _NC + c
        row_base = w * (per_w // _SL)
        iota = lax.broadcasted_iota(jnp.int32, (16,), 0)
        zf = jnp.zeros((16,), jnp.float32)

        def init_b(z, carry):
            for b in range(nb):
                st_v[pl.ds(b * st_len + z * 16, 16)] = zf
                st_c[pl.ds(b * st_len + z * 16, 16)] = (iota + z) & 8191
                st_s[pl.ds(b * st_len + z * 16, 16)] = (
                    jnp.zeros((16,), jnp.int32) + b * _GCHUNK
                    if bucket_shift == 0 else
                    jnp.zeros((16,), jnp.int32) + b * (1 << bucket_shift))
            return carry
        lax.fori_loop(0, st_len // 16, init_b, 0)
        for b in range(2 * nb):
            offs[b] = 0

        def flush_block(b, off):
            tot = pl.multiple_of(offs[nb + b], _SB)
            pltpu.sync_copy(st_c.at[pl.ds(b * st_len, _SB)],
                            cols_o.at[w, b, pl.ds(tot, _SB)])
            pltpu.sync_copy(st_s.at[pl.ds(b * st_len, _SB)],
                            segs_o.at[w, b, pl.ds(tot, _SB)])
            pltpu.sync_copy(st_v.at[pl.ds(b * st_len, _SB)],
                            vals_o.at[w, b, pl.ds(tot, _SB)])
            mc = st_c[pl.ds(b * st_len + _SB, 16)]
            ms = st_s[pl.ds(b * st_len + _SB, 16)]
            mv = st_v[pl.ds(b * st_len + _SB, 16)]
            rem = off - _SB
            mv = jnp.where(iota < rem, mv, 0.0)

            def zv(z, cz):
                st_v[pl.ds(b * st_len + z * 16, 16)] = zf
                return cz
            lax.fori_loop(0, st_len // 16, zv, 0)
            st_c[pl.ds(b * st_len, 16)] = mc
            st_s[pl.ds(b * st_len, 16)] = ms
            st_v[pl.ds(b * st_len, 16)] = mv
            offs[b] = rem
            offs[nb + b] = tot + _SB

        def sb_body(sb, carry):
            r0 = row_base + sb * _KS
            pltpu.sync_copy(cols2.at[pl.ds(r0, _KS)], colv)
            pltpu.sync_copy(segs2.at[pl.ds(r0, _KS)], segv)
            pltpu.sync_copy(vals2.at[pl.ds(r0, _KS)], valv)

            def grp(g, carry2):
                gj = g >> 3
                gq = (g & 7) * 16
                sg = segv[gj, pl.ds(gq, 16)]
                cg = colv[gj, pl.ds(gq, 16)]
                vg = valv[gj, pl.ds(gq, 16)]
                if bucket_shift == 0:
                    bv = (sg >= _GCHUNK).astype(jnp.int32)
                else:
                    bv = jnp.right_shift(sg, jnp.int32(bucket_shift))
                for b in range(nb):
                    off = offs[b]

                    @pl.when(off >= _SB)
                    def _(b=b, off=off):
                        flush_block(b, off)

                    off2 = offs[b]
                    m = bv == b
                    plsc.store_compressed(
                        st_c.at[pl.ds(b * st_len + off2, 16)], cg, mask=m)
                    plsc.store_compressed(
                        st_s.at[pl.ds(b * st_len + off2, 16)], sg, mask=m)
                    plsc.store_compressed(
                        st_v.at[pl.ds(b * st_len + off2, 16)], vg, mask=m)
                    pc = plsc.all_reduce_population_count(m)
                    offs[b] = off2 + pc[0]
                return carry2

            lax.fori_loop(0, _KS * _SL // 16, grp, 0)
            return carry

        lax.fori_loop(0, per_w // _SB, sb_body, 0)

        for b in range(nb):
            offp = offs[b]

            @pl.when(offp >= _SB)
            def _(b=b, offp=offp):
                flush_block(b, offp)

        cz16 = jnp.zeros((16,), jnp.int32)
        clo = cz16
        chi = cz16
        for b in range(nb):
            tot = pl.multiple_of(offs[nb + b], _SB)
            pltpu.sync_copy(st_c.at[pl.ds(b * st_len, _SB)],
                            cols_o.at[w, b, pl.ds(tot, _SB)])
            pltpu.sync_copy(st_s.at[pl.ds(b * st_len, _SB)],
                            segs_o.at[w, b, pl.ds(tot, _SB)])
            pltpu.sync_copy(st_v.at[pl.ds(b * st_len, _SB)],
                            vals_o.at[w, b, pl.ds(tot, _SB)])
            cnt = tot + offs[b]
            if b < 16:
                clo = jnp.where(iota == b, cnt, clo)
            else:
                chi = jnp.where(iota == (b - 16), cnt, chi)
        cnt_v[pl.ds(0, 16)] = clo
        cnt_v[pl.ds(16, 16)] = chi
        pltpu.sync_copy(cnt_v, counts_o.at[w])

    return part


def _scale_rows(rows_v, val_v, j):
    def body(q, carry):
        vv = val_v[j, pl.ds(q * 16, 16)]
        base = q * 16
        for e in range(16):
            v = vv[e]
            for k in range(_D // 16):
                sl = pl.ds(k * 16, 16)
                rows_v[j, base + e, sl] = rows_v[j, base + e, sl] * v
        return carry
    lax.fori_loop(0, _SL // 16, body, 0)


def _make_spmm(e_pad, chunk, cpc, nb, out_rows):
    cap = e_pad // _NW + _SB
    wz = ((chunk // _NS) + 7) // 8 * 8
    wz_last = chunk - 15 * wz

    mesh = plsc.VectorSubcoreMesh(core_axis_name="c", subcore_axis_name="s")

    @functools.partial(
        pl.kernel,
        out_type=jax.ShapeDtypeStruct((out_rows, _D), jnp.float32),
        mesh=mesh,
        compiler_params=pltpu.CompilerParams(
            use_tc_tiling_on_sc=False, needs_layout_passes=False),
        scratch_types=[
            pltpu.VMEM((_KS, _SL), jnp.int32),
            pltpu.VMEM((_KS, _SL), jnp.int32),
            pltpu.VMEM((_KS, _SL), jnp.float32),
            pltpu.VMEM((_KS, _SL, _D), jnp.float32),
            pltpu.VMEM((64,), jnp.int32),
            pltpu.VMEM_SHARED((chunk, _D), jnp.float32),
            pltpu.SemaphoreType.DMA,
            pltpu.SemaphoreType.DMA,
            pltpu.SemaphoreType.DMA,
        ],
    )
    def spmm(table, cols5, segs5, vals5, counts, zeros, out, idx_v, seg_v,
             val_v, rows_v, cv, acc, sem, sem2, sem3):
        c = lax.axis_index("c")
        s = lax.axis_index("s")
        iota = lax.broadcasted_iota(jnp.int32, (16,), 0)
        pltpu.sync_copy(counts.at[2 * s], cv.at[pl.ds(0, 32)])
        pltpu.sync_copy(counts.at[2 * s + 1], cv.at[pl.ds(32, 32)])

        def one_bucket(bl, carry):
            b = c * cpc + bl

            @pl.when(s < 15)
            def _():
                pltpu.sync_copy(zeros.at[pl.ds(s * wz, wz)],
                                acc.at[pl.ds(s * wz, wz)])

            @pl.when(s == 15)
            def _():
                pltpu.sync_copy(zeros.at[pl.ds(15 * wz, wz_last)],
                                acc.at[pl.ds(15 * wz, wz_last)])

            plsc.subcore_barrier()
            base = b * chunk

            for wi in range(2):
                w = 2 * s + wi
                lo = cv[pl.ds(wi * 32, 16)]
                if cpc == 1:
                    cnt = jnp.where(c == 0, lo[0], lo[1])
                else:
                    cnt = jnp.sum(jnp.where(iota == b, lo, 0))
                    if nb > 16:
                        hi = cv[pl.ds(wi * 32 + 16, 16)]
                        cnt = cnt + jnp.sum(
                            jnp.where(iota == (b - 16), hi, 0))
                nsb = (cnt + _SB - 1) >> 10

                def sb_body(sb, carry2):
                    @pl.when(sb < nsb)
                    def _():
                        r0 = ((w * nb + b) * (cap // _SL)) + sb * _KS
                        pltpu.sync_copy(cols5.at[pl.ds(r0, _KS)], idx_v)
                        cps = [pltpu.async_copy(table.at[idx_v.at[j]],
                                                rows_v.at[j], sem)
                               for j in range(_KS)]
                        ld_s = pltpu.async_copy(segs5.at[pl.ds(r0, _KS)],
                                                seg_v, sem2)
                        ld_v = pltpu.async_copy(vals5.at[pl.ds(r0, _KS)],
                                                val_v, sem2)
                        ld_s.wait()
                        ld_v.wait()
                        sc_cps = []
                        for j in range(_KS):
                            cps[j].wait()
                            _scale_rows(rows_v, val_v, j)
                            for q in range(_SL // 16):
                                sl = pl.ds(q * 16, 16)
                                seg_v[j, sl] = seg_v[j, sl] - base
                            sc_cps.append(pltpu.async_copy(
                                rows_v.at[j], acc.at[seg_v.at[j]], sem3,
                                add=True))
                        for cp in sc_cps:
                            cp.wait()
                    return carry2

                lax.fori_loop(0, cap // _SB, sb_body, 0)

            plsc.subcore_barrier()

            @pl.when(s < 15)
            def _():
                pltpu.sync_copy(acc.at[pl.ds(s * wz, wz)],
                                out.at[pl.ds(base + s * wz, wz)])

            @pl.when(s == 15)
            def _():
                pltpu.sync_copy(acc.at[pl.ds(15 * wz, wz_last)],
                                out.at[pl.ds(base + 15 * wz, wz_last)])

            plsc.subcore_barrier()
            return carry

        lax.fori_loop(0, cpc, one_bucket, 0)

    return spmm


def _dense_layer(um, im, init_g, wq1, bq1, wq2, wu, bu, wi, bi):
    bg = 2000
    f32 = jnp.float32

    def body(um_ref, im_ref, g_ref, wq1_ref, bq1_ref, wq2_ref, wu_ref,
             bu_ref, wi_ref, bi_ref, msg_ref):
        u = um_ref[...]
        i = im_ref[...]
        w1 = wq1_ref[...]
        b1 = bq1_ref[...]
        w2 = wq2_ref[...]
        tu = jnp.tanh(jnp.dot(u, w1, preferred_element_type=f32) + b1)
        ti = jnp.tanh(jnp.dot(i, w1, preferred_element_type=f32) + b1)
        au = jnp.sum(tu * w2, axis=1, keepdims=True)
        ai = jnp.sum(ti * w2, axis=1, keepdims=True)
        w0 = jax.nn.sigmoid(au - ai)
        common = w0 * u + (1.0 - w0) * i
        g = g_ref[...]
        mu = (jnp.dot(u - common, wu_ref[0:_D, :], preferred_element_type=f32)
              + jnp.dot(g, wu_ref[_D:2 * _D, :], preferred_element_type=f32)
              + bu_ref[...])
        mi = (jnp.dot(i - common, wi_ref[0:_D, :], preferred_element_type=f32)
              + jnp.dot(g, wi_ref[_D:2 * _D, :], preferred_element_type=f32)
              + bi_ref[...])
        msg_ref[...] = mu + mi + common

    def wspec(shape):
        return pl.BlockSpec(shape, lambda i: tuple(0 for _ in shape))

    gspec = pl.BlockSpec((bg, _D), lambda i: (i, 0))
    return pl.pallas_call(
        body,
        grid=(_G // bg,),
        in_specs=[
            gspec, gspec, gspec,
            wspec((_D, _D)), wspec((1, _D)), wspec((1, _D)),
            wspec((2 * _D, _D)), wspec((1, _D)),
            wspec((2 * _D, _D)), wspec((1, _D)),
        ],
        out_specs=gspec,
        out_shape=jax.ShapeDtypeStruct((_G, _D), jnp.float32),
    )(um, im, init_g, wq1, bq1, wq2, wu, bu, wi, bi)


def _add4(rows, bs):
    def body(a_ref, b_ref, c_ref, d_ref, o_ref):
        o_ref[...] = a_ref[...] + b_ref[...] + c_ref[...] + d_ref[...]

    spec = pl.BlockSpec((bs, _D), lambda i: (i, 0))
    return pl.pallas_call(
        body,
        grid=(rows // bs,),
        in_specs=[spec, spec, spec, spec],
        out_specs=spec,
        out_shape=jax.ShapeDtypeStruct((rows, _D), jnp.float32),
    )


def _prep_edges(segs, cols, vals, e_pad, col_mod, col_off):
    e = segs.shape[0]
    pad = e_pad - e
    pidx = jnp.arange(pad, dtype=jnp.int32)
    segs_p = jnp.concatenate([segs, pidx % jnp.int32(8192)])
    cols_p = jnp.concatenate(
        [cols + jnp.int32(col_off),
         (pidx % jnp.int32(col_mod)) + jnp.int32(col_off)])
    vals_p = jnp.concatenate([vals, jnp.zeros((pad,), jnp.float32)])
    r = e_pad // _SL
    return (cols_p.reshape(r, _SL), segs_p.reshape(r, _SL),
            vals_p.reshape(r, _SL))


def _bucketed(part_fn, cols2, segs2, vals2, e_pad, nb):
    cb, sb, vb, cnts = part_fn(cols2, segs2, vals2)
    cap = e_pad // _NW + _SB
    shp = (_NW * nb * (cap // _SL), _SL)
    return cb.reshape(shp), sb.reshape(shp), vb.reshape(shp), cnts


def kernel(user_emb, item_emb, group_emb, u_rows, u_cols, u_vals,
           i_rows, i_cols, i_vals, f_rows, f_cols, f_vals, layers):
    init_ui = jnp.concatenate(
        [user_emb, item_emb,
         jnp.zeros((_NP - _N, _D), jnp.float32)], axis=0)
    zeros = jnp.zeros((_GCHUNK, _D), jnp.float32)

    eu_p, ei_p, ef_p = _pad_up(_EU), _pad_up(_EI), _pad_up(_EF)
    u_c2, u_s2, u_v2 = _prep_edges(u_rows, u_cols, u_vals, eu_p, 65536, 0)
    i_c2, i_s2, i_v2 = _prep_edges(i_rows, i_cols, i_vals, ei_p, 32768, _U)
    f_c2, f_s2, f_v2 = _prep_edges(f_rows, f_cols, f_vals, ef_p, 8192, 0)

    u_bkt = _bucketed(_make_partition(eu_p, 2, 0), u_c2, u_s2, u_v2, eu_p, 2)
    i_bkt = _bucketed(_make_partition(ei_p, 2, 0), i_c2, i_s2, i_v2, ei_p, 2)
    f_bkt = _bucketed(_make_partition(ef_p, _NB_F, 13), f_c2, f_s2, f_v2,
                      ef_p, _NB_F)

    spmm_u = _make_spmm(eu_p, _GCHUNK, 1, 2, _G)
    spmm_i = _make_spmm(ei_p, _GCHUNK, 1, 2, _G)
    spmm_f = _make_spmm(ef_p, _FCHUNK, _NB_F // _NC, _NB_F, _NP)

    table = init_ui
    nodes, msgs = [], []
    for p in layers:
        um = spmm_u(table, *u_bkt, zeros)
        im = spmm_i(table, *i_bkt, zeros)
        msg = _dense_layer(um, im, group_emb, p["Wq1"],
                           p["bq1"].reshape(1, _D), p["Wq2"].reshape(1, _D),
                           p["Wu"], p["bu"].reshape(1, _D),
                           p["Wi"], p["bi"].reshape(1, _D))
        node = spmm_f(msg, *f_bkt, zeros)
        nodes.append(node)
        msgs.append(msg)
        table = node

    final_emb = _add4(_NP, 2560)(init_ui, *nodes)
    he_emb = _add4(_G, 2000)(group_emb, *msgs)
    return jnp.concatenate([final_emb[:_N], he_emb], axis=0)
